# baseline (speedup 1.0000x reference)
import functools

import jax
import jax.numpy as jnp
from jax.experimental import pallas as pl
from jax.experimental.pallas import tpu as pltpu

B = 8
EPS = 1e-5
CHUNK = 512

_DN_T = (((0,), (0,)), ((), ()))


def _onehot_t(lo_ref, hi_ref, row0, rows, nseg, dtype=jnp.float32):
    r2 = jax.lax.broadcasted_iota(jnp.int32, (nseg, rows), 1) + row0
    lo = lo_ref[:, 0:1]
    hi = hi_ref[:, 0:1]
    return ((r2 >= lo) & (r2 < hi)).astype(dtype)


def _split_bf16(a):
    hi = a.astype(jnp.bfloat16)
    lo = (a - hi.astype(jnp.float32)).astype(jnp.bfloat16)
    return jnp.concatenate([hi, lo], axis=0)


def _seg_of(cu_ref, row):
    seg = jnp.int32(0)
    for s in range(1, B):
        seg += (row >= cu_ref[s]).astype(jnp.int32)
    return seg


def _fused_kernel(cu_ref, x_ref, lo_ref, hi_ref, gamma_ref, beta_ref, wt_ref,
                  wtbf_ref, b_ref, y_ref, xc_ref, sums_ref, sq_ref, scale_ref,
                  bias2_ref, bias2f_ref, scalef_ref, *, rows, nblocks):
    p = pl.program_id(0)
    i = pl.program_id(1)

    @pl.when(p == 0)
    def _stats():
        @pl.when(i == 0)
        def _init():
            sums_ref[...] = jnp.zeros_like(sums_ref)
            sq_ref[...] = jnp.zeros_like(sq_ref)

        xb = x_ref[...].astype(jnp.bfloat16)
        xc_ref[pl.ds(i * rows, rows), :] = xb
        ohT = _onehot_t(lo_ref[0:B], hi_ref[0:B], i * rows, rows, B,
                        jnp.bfloat16)
        sums_ref[...] += jax.lax.dot(ohT, xb,
                                     preferred_element_type=jnp.float32)
        sq_ref[...] += jax.lax.dot(ohT, xb * xb,
                                   preferred_element_type=jnp.float32)

        @pl.when(i == nblocks - 1)
        def _finalize():
            cnts = []
            for s in range(B):
                cnts.append((cu_ref[s + 1] - cu_ref[s]).astype(jnp.float32))
            counts = jnp.stack(cnts).reshape(B, 1)
            mean = sums_ref[...] / counts
            var = sq_ref[...] / counts - mean * mean
            scale = gamma_ref[...] * jax.lax.rsqrt(var + EPS)
            bias = beta_ref[...] - mean * scale
            bias2 = (jax.lax.dot(bias, wt_ref[...],
                                 preferred_element_type=jnp.float32)
                     + b_ref[...])
            bias2f_ref[...] = bias2
            scalef_ref[...] = scale
            scale_ref[...] = _split_bf16(scale)
            bias2_ref[...] = _split_bf16(bias2)

    @pl.when(p == 1)
    def _apply():
        r0 = i * rows
        seg0 = _seg_of(cu_ref, r0)
        seg1 = _seg_of(cu_ref, r0 + rows - 1)

        @pl.when(seg0 == seg1)
        def _uniform():
            s_row = scalef_ref[pl.ds(seg0, 1), :].astype(jnp.bfloat16)
            xs = xc_ref[pl.ds(r0, rows), :] * s_row
            y_ref[...] = (
                jax.lax.dot(xs, wtbf_ref[...],
                            preferred_element_type=jnp.float32)
                + bias2f_ref[pl.ds(seg0, 1), :])

        @pl.when(seg0 != seg1)
        def _mixed():
            for k in range(rows // CHUNK):
                oh2 = _onehot_t(lo_ref, hi_ref, r0 + k * CHUNK, CHUNK, 2 * B,
                                jnp.bfloat16)
                scale_b = jax.lax.dot_general(
                    oh2, scale_ref[...], _DN_T,
                    preferred_element_type=jnp.float32
                ).astype(jnp.bfloat16)
                bias_b = jax.lax.dot_general(
                    oh2, bias2_ref[...], _DN_T,
                    preferred_element_type=jnp.float32)
                xs = xc_ref[pl.ds(r0 + k * CHUNK, CHUNK), :] * scale_b
                y_ref[pl.ds(k * CHUNK, CHUNK), :] = (
                    jax.lax.dot(xs, wtbf_ref[...],
                                preferred_element_type=jnp.float32) + bias_b)


@jax.jit
def kernel(flat_features, cu_seqlens, gamma, beta, W, b):
    T, C = flat_features.shape
    WT = W.T
    WTbf = WT.astype(jnp.bfloat16)
    gamma2 = gamma.reshape(1, C)
    beta2 = beta.reshape(1, C)
    b2 = b.reshape(1, C)
    lo1 = jnp.broadcast_to(cu_seqlens[:B, None], (B, 128)).astype(jnp.int32)
    hi1 = jnp.broadcast_to(cu_seqlens[1:B + 1, None], (B, 128)).astype(jnp.int32)
    lo_b = jnp.concatenate([lo1, lo1], axis=0)
    hi_b = jnp.concatenate([hi1, hi1], axis=0)

    rows = 2048
    nb = T // rows
    grid_spec = pltpu.PrefetchScalarGridSpec(
        num_scalar_prefetch=1,
        grid=(2, nb),
        in_specs=[
            pl.BlockSpec((rows, C),
                         lambda p, i, cu: (i * (1 - p) + (nb - 1) * p, 0)),
            pl.BlockSpec((2 * B, 128), lambda p, i, cu: (0, 0)),
            pl.BlockSpec((2 * B, 128), lambda p, i, cu: (0, 0)),
            pl.BlockSpec((1, C), lambda p, i, cu: (0, 0)),
            pl.BlockSpec((1, C), lambda p, i, cu: (0, 0)),
            pl.BlockSpec((C, C), lambda p, i, cu: (0, 0)),
            pl.BlockSpec((C, C), lambda p, i, cu: (0, 0)),
            pl.BlockSpec((1, C), lambda p, i, cu: (0, 0)),
        ],
        out_specs=pl.BlockSpec((rows, C), lambda p, i, cu: (i * p, 0)),
        scratch_shapes=[
            pltpu.VMEM((T, C), jnp.bfloat16),
            pltpu.VMEM((B, C), jnp.float32),
            pltpu.VMEM((B, C), jnp.float32),
            pltpu.VMEM((2 * B, C), jnp.bfloat16),
            pltpu.VMEM((2 * B, C), jnp.bfloat16),
            pltpu.VMEM((B, C), jnp.float32),
            pltpu.VMEM((B, C), jnp.float32),
        ],
    )
    y = pl.pallas_call(
        functools.partial(_fused_kernel, rows=rows, nblocks=nb),
        grid_spec=grid_spec,
        out_shape=jax.ShapeDtypeStruct((T, C), jnp.float32),
        compiler_params=pltpu.CompilerParams(
            dimension_semantics=("arbitrary", "arbitrary")),
    )(cu_seqlens, flat_features, lo_b, hi_b, gamma2, beta2, WT, WTbf, b2)
    return y


# speedup vs baseline: 1.0052x; 1.0052x over previous
import functools

import jax
import jax.numpy as jnp
from jax.experimental import pallas as pl
from jax.experimental.pallas import tpu as pltpu

B = 8
EPS = 1e-5
CHUNK = 512

_DN_T = (((0,), (0,)), ((), ()))


def _onehot_t(lo_ref, hi_ref, row0, rows, nseg, dtype=jnp.float32):
    r2 = jax.lax.broadcasted_iota(jnp.int32, (nseg, rows), 1) + row0
    lo = lo_ref[:, 0:1]
    hi = hi_ref[:, 0:1]
    return ((r2 >= lo) & (r2 < hi)).astype(dtype)


def _split_bf16(a):
    hi = a.astype(jnp.bfloat16)
    lo = (a - hi.astype(jnp.float32)).astype(jnp.bfloat16)
    return jnp.concatenate([hi, lo], axis=0)


def _seg_of(cu_ref, row):
    seg = jnp.int32(0)
    for s in range(1, B):
        seg += (row >= cu_ref[s]).astype(jnp.int32)
    return seg


def _fused_kernel(cu_ref, x_ref, lo_ref, hi_ref, gamma_ref, beta_ref, wt_ref,
                  wtbf_ref, b_ref, y_ref, xc_ref, sums_ref, sq_ref, scale_ref,
                  bias2_ref, bias2f_ref, scalef_ref, *, rows, nblocks):
    p = pl.program_id(0)
    i = pl.program_id(1)

    @pl.when(p == 0)
    def _stats():
        @pl.when(i == 0)
        def _init():
            sums_ref[...] = jnp.zeros_like(sums_ref)
            sq_ref[...] = jnp.zeros_like(sq_ref)

        x = x_ref[...]
        xc_ref[pl.ds(i * rows, rows), :] = x.astype(jnp.bfloat16)
        ohT = _onehot_t(lo_ref[0:B], hi_ref[0:B], i * rows, rows, B)
        sums_ref[...] += jax.lax.dot(ohT, x, preferred_element_type=jnp.float32)
        sq_ref[...] += jax.lax.dot(ohT, x * x,
                                   preferred_element_type=jnp.float32)

        @pl.when(i == nblocks - 1)
        def _finalize():
            cnts = []
            for s in range(B):
                cnts.append((cu_ref[s + 1] - cu_ref[s]).astype(jnp.float32))
            counts = jnp.stack(cnts).reshape(B, 1)
            mean = sums_ref[...] / counts
            var = sq_ref[...] / counts - mean * mean
            scale = gamma_ref[...] * jax.lax.rsqrt(var + EPS)
            bias = beta_ref[...] - mean * scale
            bias2 = (jax.lax.dot(bias, wt_ref[...],
                                 preferred_element_type=jnp.float32)
                     + b_ref[...])
            bias2f_ref[...] = bias2
            scalef_ref[...] = scale
            scale_ref[...] = _split_bf16(scale)
            bias2_ref[...] = _split_bf16(bias2)

    @pl.when(p == 1)
    def _apply():
        r0 = i * rows
        seg0 = _seg_of(cu_ref, r0)
        seg1 = _seg_of(cu_ref, r0 + rows - 1)

        @pl.when(seg0 == seg1)
        def _uniform():
            s_row = scalef_ref[pl.ds(seg0, 1), :].astype(jnp.bfloat16)
            xs = xc_ref[pl.ds(r0, rows), :] * s_row
            y_ref[...] = (
                jax.lax.dot(xs, wtbf_ref[...],
                            preferred_element_type=jnp.float32)
                + bias2f_ref[pl.ds(seg0, 1), :])

        @pl.when(seg0 != seg1)
        def _mixed():
            for k in range(rows // CHUNK):
                oh2 = _onehot_t(lo_ref, hi_ref, r0 + k * CHUNK, CHUNK, 2 * B,
                                jnp.bfloat16)
                scale_b = jax.lax.dot_general(
                    oh2, scale_ref[...], _DN_T,
                    preferred_element_type=jnp.float32
                ).astype(jnp.bfloat16)
                bias_b = jax.lax.dot_general(
                    oh2, bias2_ref[...], _DN_T,
                    preferred_element_type=jnp.float32)
                xs = xc_ref[pl.ds(r0 + k * CHUNK, CHUNK), :] * scale_b
                y_ref[pl.ds(k * CHUNK, CHUNK), :] = (
                    jax.lax.dot(xs, wtbf_ref[...],
                                preferred_element_type=jnp.float32) + bias_b)


@jax.jit
def kernel(flat_features, cu_seqlens, gamma, beta, W, b):
    T, C = flat_features.shape
    WT = W.T
    WTbf = WT.astype(jnp.bfloat16)
    gamma2 = gamma.reshape(1, C)
    beta2 = beta.reshape(1, C)
    b2 = b.reshape(1, C)
    lo1 = jnp.broadcast_to(cu_seqlens[:B, None], (B, 128)).astype(jnp.int32)
    hi1 = jnp.broadcast_to(cu_seqlens[1:B + 1, None], (B, 128)).astype(jnp.int32)
    lo_b = jnp.concatenate([lo1, lo1], axis=0)
    hi_b = jnp.concatenate([hi1, hi1], axis=0)

    rows = 2048
    nb = T // rows
    grid_spec = pltpu.PrefetchScalarGridSpec(
        num_scalar_prefetch=1,
        grid=(2, nb),
        in_specs=[
            pl.BlockSpec((rows, C),
                         lambda p, i, cu: (i * (1 - p) + (nb - 1) * p, 0)),
            pl.BlockSpec((2 * B, 128), lambda p, i, cu: (0, 0)),
            pl.BlockSpec((2 * B, 128), lambda p, i, cu: (0, 0)),
            pl.BlockSpec((1, C), lambda p, i, cu: (0, 0)),
            pl.BlockSpec((1, C), lambda p, i, cu: (0, 0)),
            pl.BlockSpec((C, C), lambda p, i, cu: (0, 0)),
            pl.BlockSpec((C, C), lambda p, i, cu: (0, 0)),
            pl.BlockSpec((1, C), lambda p, i, cu: (0, 0)),
        ],
        out_specs=pl.BlockSpec((rows, C), lambda p, i, cu: (i * p, 0)),
        scratch_shapes=[
            pltpu.VMEM((T, C), jnp.bfloat16),
            pltpu.VMEM((B, C), jnp.float32),
            pltpu.VMEM((B, C), jnp.float32),
            pltpu.VMEM((2 * B, C), jnp.bfloat16),
            pltpu.VMEM((2 * B, C), jnp.bfloat16),
            pltpu.VMEM((B, C), jnp.float32),
            pltpu.VMEM((B, C), jnp.float32),
        ],
    )
    y = pl.pallas_call(
        functools.partial(_fused_kernel, rows=rows, nblocks=nb),
        grid_spec=grid_spec,
        out_shape=jax.ShapeDtypeStruct((T, C), jnp.float32),
        compiler_params=pltpu.CompilerParams(
            dimension_semantics=("arbitrary", "arbitrary")),
    )(cu_seqlens, flat_features, lo_b, hi_b, gamma2, beta2, WT, WTbf, b2)
    return y
